# baseline (speedup 1.0000x reference)
import jax
import jax.numpy as jnp
from jax import lax
from jax.experimental import pallas as pl
from jax.experimental.pallas import tpu as pltpu
from jax.experimental.pallas import tpu_sc as plsc

VOCAB = 1000000
EMBED_DIM = 64
OUT_DIM = 10
POUT = 16
BATCH = 16384
FIELDS = 26
NUM_WORKERS = 32
XROWS_PER_W = BATCH // NUM_WORKERS
G = 16
NG = XROWS_PER_W // G
FIRE = 4
GROUPS = NG // FIRE

PROJ_BLK = 8000
PROJ_GRID = VOCAB // PROJ_BLK


def _proj_body(x_ref, w_ref, b_ref, o_ref):
    acc = b_ref[...]
    for k in range(8):
        xk = x_ref[pl.Slice(k, PROJ_BLK // 8, 8), :]
        acc = acc + jnp.dot(
            xk,
            w_ref[pl.Slice(k * EMBED_DIM, EMBED_DIM), :],
            preferred_element_type=jnp.float32,
        )
    o_ref[...] = acc


_tc_project = pl.pallas_call(
    _proj_body,
    grid=(PROJ_GRID,),
    in_specs=[
        pl.BlockSpec((PROJ_BLK, EMBED_DIM), lambda i: (i, 0)),
        pl.BlockSpec((8 * EMBED_DIM, 8 * POUT), lambda i: (0, 0)),
        pl.BlockSpec((1, 8 * POUT), lambda i: (0, 0)),
    ],
    out_specs=pl.BlockSpec((PROJ_BLK // 8, 8 * POUT), lambda i: (i, 0)),
    out_shape=jax.ShapeDtypeStruct((VOCAB // 8, 8 * POUT), jnp.float32),
    compiler_params=pltpu.CompilerParams(
        dimension_semantics=("parallel",),
    ),
)


GX = 8
NGROUPS = XROWS_PER_W // GX
HALF = NGROUPS // 2


def _fire(tab_hbm, idx_v, buf, sem, grp):
    copies = []
    for j in range(GX):
        copies.append(
            pltpu.async_copy(
                tab_hbm.at[idx_v.at[grp * GX + j]],
                buf.at[j],
                sem,
            )
        )
    return copies


def _drain_out(out_hbm, buf, copies, xbase, grp):
    for c in copies:
        c.wait()
    pltpu.sync_copy(
        buf,
        out_hbm.at[pl.ds(xbase + grp * GX, GX)],
    )


def _gather_body(tab_hbm, xid_hbm, out_hbm, idx_v, buf_a, buf_b, sem):
    wid = lax.axis_index("s") * 2 + lax.axis_index("c")
    xbase = wid * XROWS_PER_W
    pltpu.sync_copy(xid_hbm.at[pl.ds(xbase, XROWS_PER_W)], idx_v)

    ca0 = _fire(tab_hbm, idx_v, buf_a, sem, 0)

    def iteration(h, carry):
        cb = _fire(tab_hbm, idx_v, buf_b, sem, 2 * h + 1)
        ca = [pltpu.make_async_copy(tab_hbm.at[idx_v.at[2 * h * GX + j]],
                                    buf_a.at[j], sem) for j in range(GX)]
        _drain_out(out_hbm, buf_a, ca, xbase, 2 * h)

        @pl.when(h < HALF - 1)
        def _():
            _fire(tab_hbm, idx_v, buf_a, sem, 2 * h + 2)

        _drain_out(out_hbm, buf_b, cb, xbase, 2 * h + 1)
        return carry

    lax.fori_loop(0, HALF, iteration, 0)
    del ca0


_sc_gather = pl.kernel(
    _gather_body,
    out_type=jax.ShapeDtypeStruct((BATCH, FIELDS, POUT), jnp.float32),
    mesh=plsc.VectorSubcoreMesh(core_axis_name="c", subcore_axis_name="s"),
    scratch_types=[
        pltpu.VMEM((XROWS_PER_W, FIELDS), jnp.int32),
        pltpu.VMEM((GX, FIELDS, POUT), jnp.float32),
        pltpu.VMEM((GX, FIELDS, POUT), jnp.float32),
        pltpu.SemaphoreType.DMA,
    ],
    compiler_params=pltpu.CompilerParams(use_tc_tiling_on_sc=False),
)


def _sc_gather_fn(tab, xid):
    return _sc_gather(tab, xid)


def kernel(x_id, emb, W, b):
    wtp = jnp.concatenate(
        [W.T, jnp.zeros((EMBED_DIM, POUT - OUT_DIM), jnp.float32)], axis=1
    )
    bp = jnp.concatenate(
        [b, jnp.zeros((POUT - OUT_DIM,), jnp.float32)]
    ).reshape(1, POUT)
    wtp = jax.scipy.linalg.block_diag(*([wtp] * 8))
    bp = jnp.tile(bp, (1, 8))
    tab = _tc_project(emb, wtp, bp).reshape(VOCAB, POUT)
    padded = _sc_gather_fn(tab, x_id.astype(jnp.int32))
    return padded[..., :OUT_DIM]


# speedup vs baseline: 1.0667x; 1.0667x over previous
import jax
import jax.numpy as jnp
from jax import lax
from jax.experimental import pallas as pl
from jax.experimental.pallas import tpu as pltpu
from jax.experimental.pallas import tpu_sc as plsc

VOCAB = 1000000
EMBED_DIM = 64
OUT_DIM = 10
POUT = 16
BATCH = 16384
FIELDS = 26
NUM_WORKERS = 32
XROWS_PER_W = BATCH // NUM_WORKERS
G = 16
NG = XROWS_PER_W // G
FIRE = 4
GROUPS = NG // FIRE

PROJ_BLK = 40000
PROJ_GRID = VOCAB // PROJ_BLK


def _proj_body(x_ref, w_ref, b_ref, o_ref):
    acc = b_ref[...]
    for k in range(8):
        xk = x_ref[pl.Slice(k, PROJ_BLK // 8, 8), :]
        acc = acc + jnp.dot(
            xk,
            w_ref[pl.Slice(k * EMBED_DIM, EMBED_DIM), :],
            preferred_element_type=jnp.float32,
        )
    o_ref[...] = acc


_tc_project = pl.pallas_call(
    _proj_body,
    grid=(PROJ_GRID,),
    in_specs=[
        pl.BlockSpec((PROJ_BLK, EMBED_DIM), lambda i: (i, 0)),
        pl.BlockSpec((8 * EMBED_DIM, 8 * POUT), lambda i: (0, 0)),
        pl.BlockSpec((1, 8 * POUT), lambda i: (0, 0)),
    ],
    out_specs=pl.BlockSpec((PROJ_BLK // 8, 8 * POUT), lambda i: (i, 0)),
    out_shape=jax.ShapeDtypeStruct((VOCAB // 8, 8 * POUT), jnp.float32),
    compiler_params=pltpu.CompilerParams(
        dimension_semantics=("parallel",),
    ),
)


GX = 8
NGROUPS = XROWS_PER_W // GX
HALF = NGROUPS // 2


def _fire(tab_hbm, idx_v, buf, sem, grp):
    copies = []
    for j in range(GX):
        copies.append(
            pltpu.async_copy(
                tab_hbm.at[idx_v.at[grp * GX + j]],
                buf.at[j],
                sem,
            )
        )
    return copies


def _drain_out(out_hbm, buf, copies, xbase, grp):
    for c in copies:
        c.wait()
    pltpu.sync_copy(
        buf,
        out_hbm.at[pl.ds(xbase + grp * GX, GX)],
    )


def _gather_body(tab_hbm, xid_hbm, out_hbm, idx_v, buf_a, buf_b, sem):
    wid = lax.axis_index("s") * 2 + lax.axis_index("c")
    xbase = wid * XROWS_PER_W
    pltpu.sync_copy(xid_hbm.at[pl.ds(xbase, XROWS_PER_W)], idx_v)

    ca0 = _fire(tab_hbm, idx_v, buf_a, sem, 0)

    def iteration(h, carry):
        cb = _fire(tab_hbm, idx_v, buf_b, sem, 2 * h + 1)
        ca = [pltpu.make_async_copy(tab_hbm.at[idx_v.at[2 * h * GX + j]],
                                    buf_a.at[j], sem) for j in range(GX)]
        _drain_out(out_hbm, buf_a, ca, xbase, 2 * h)

        @pl.when(h < HALF - 1)
        def _():
            _fire(tab_hbm, idx_v, buf_a, sem, 2 * h + 2)

        _drain_out(out_hbm, buf_b, cb, xbase, 2 * h + 1)
        return carry

    lax.fori_loop(0, HALF, iteration, 0)
    del ca0


_sc_gather = pl.kernel(
    _gather_body,
    out_type=jax.ShapeDtypeStruct((BATCH, FIELDS, POUT), jnp.float32),
    mesh=plsc.VectorSubcoreMesh(core_axis_name="c", subcore_axis_name="s"),
    scratch_types=[
        pltpu.VMEM((XROWS_PER_W, FIELDS), jnp.int32),
        pltpu.VMEM((GX, FIELDS, POUT), jnp.float32),
        pltpu.VMEM((GX, FIELDS, POUT), jnp.float32),
        pltpu.SemaphoreType.DMA,
    ],
    compiler_params=pltpu.CompilerParams(use_tc_tiling_on_sc=False),
)


def _sc_gather_fn(tab, xid):
    return _sc_gather(tab, xid)


def kernel(x_id, emb, W, b):
    wtp = jnp.concatenate(
        [W.T, jnp.zeros((EMBED_DIM, POUT - OUT_DIM), jnp.float32)], axis=1
    )
    bp = jnp.concatenate(
        [b, jnp.zeros((POUT - OUT_DIM,), jnp.float32)]
    ).reshape(1, POUT)
    wtp = jax.scipy.linalg.block_diag(*([wtp] * 8))
    bp = jnp.tile(bp, (1, 8))
    tab = _tc_project(emb, wtp, bp).reshape(VOCAB, POUT)
    padded = _sc_gather_fn(tab, x_id.astype(jnp.int32))
    return padded[..., :OUT_DIM]
